# Initial kernel scaffold; baseline (speedup 1.0000x reference)
import jax
import jax.numpy as jnp
from jax.experimental import pallas as pl


def kernel(x, edge_index, TRAIN, W1, b1, W2, b2, W3, b3, fcW1, fcb1, fcW2, fcb2, fcW3, fcb3):
    raise NotImplementedError("write your pallas kernel here")


# speedup vs baseline: 15.9361x; 15.9361x over previous
import functools

import jax
import jax.numpy as jnp
from jax import lax
from jax.experimental import pallas as pl
from jax.experimental.pallas import tpu as pltpu
from jax.experimental.pallas import tpu_sc as plsc

_N = 10000
_E = 320000
_D = 128
_C = 121

_NC = 2
_NS = 16
_NW = _NC * _NS
_EW = _E // _NW
_K = 80
_NCHUNK = _EW // _K
_NP = 10240
_RPT = _NP // _NS

_mesh = plsc.VectorSubcoreMesh(core_axis_name="c", subcore_axis_name="s")


@functools.partial(
    pl.kernel,
    out_type=jax.ShapeDtypeStruct((_NC, _NS, _RPT, 16), jnp.float32),
    mesh=_mesh,
    scratch_types=[
        pltpu.VMEM((_NCHUNK, _K), jnp.int32),
        pltpu.VMEM((_K, 16), jnp.float32),
        pltpu.VMEM_SHARED((_NP, 16), jnp.float32),
    ],
)
def _deg_pass(dst_hbm, ones_hbm, zeros_hbm, out_hbm, dst_v, ones_v, deg_sh):
    c = lax.axis_index("c")
    s = lax.axis_index("s")
    wid = s * _NC + c
    rows = pl.ds(s * _RPT, _RPT)
    pltpu.sync_copy(zeros_hbm, deg_sh.at[rows])
    pltpu.sync_copy(dst_hbm.at[wid], dst_v)
    pltpu.sync_copy(ones_hbm, ones_v)
    plsc.subcore_barrier()

    def body(j, carry):
        pltpu.sync_copy(ones_v, deg_sh.at[dst_v.at[j]], add=True)
        return carry

    lax.fori_loop(0, _NCHUNK, body, 0)
    plsc.subcore_barrier()
    pltpu.sync_copy(deg_sh.at[rows], out_hbm.at[c, s])


@functools.partial(
    pl.kernel,
    out_type=jax.ShapeDtypeStruct((_NC, _NS, _RPT, _D), jnp.float32),
    mesh=_mesh,
    scratch_types=[
        pltpu.VMEM((_NCHUNK, _K), jnp.int32),
        pltpu.VMEM((_NCHUNK, _K), jnp.int32),
        pltpu.VMEM((_K, _D), jnp.float32),
        pltpu.VMEM_SHARED((_NP, _D), jnp.float32),
    ],
)
def _prop_pass(g_hbm, src_hbm, dst_hbm, zeros_hbm, out_hbm, src_v, dst_v, rows_v, s_sh):
    c = lax.axis_index("c")
    s = lax.axis_index("s")
    wid = s * _NC + c
    rows = pl.ds(s * _RPT, _RPT)
    pltpu.sync_copy(zeros_hbm, s_sh.at[rows])
    pltpu.sync_copy(src_hbm.at[wid], src_v)
    pltpu.sync_copy(dst_hbm.at[wid], dst_v)
    plsc.subcore_barrier()

    def body(j, carry):
        pltpu.sync_copy(g_hbm.at[src_v.at[j]], rows_v)
        pltpu.sync_copy(rows_v, s_sh.at[dst_v.at[j]], add=True)
        return carry

    lax.fori_loop(0, _NCHUNK, body, 0)
    plsc.subcore_barrier()
    pltpu.sync_copy(s_sh.at[rows], out_hbm.at[c, s])


_BR = 2000
_GRID = _N // _BR


def _k0_body(x_ref, w_ref, d0_ref, d1_ref, g_ref, dis_ref):
    deg = d0_ref[...][:, :1] + d1_ref[...][:, :1] + 1.0
    dis = lax.rsqrt(deg)
    h = jnp.dot(x_ref[...], w_ref[...], preferred_element_type=jnp.float32)
    g_ref[...] = h * dis
    dis_ref[...] = jnp.broadcast_to(dis, dis_ref.shape)


_k0 = pl.pallas_call(
    _k0_body,
    grid=(_GRID,),
    in_specs=[
        pl.BlockSpec((_BR, _D), lambda i: (i, 0)),
        pl.BlockSpec((_D, _D), lambda i: (0, 0)),
        pl.BlockSpec((_BR, 16), lambda i: (i, 0)),
        pl.BlockSpec((_BR, 16), lambda i: (i, 0)),
    ],
    out_specs=[
        pl.BlockSpec((_BR, _D), lambda i: (i, 0)),
        pl.BlockSpec((_BR, 16), lambda i: (i, 0)),
    ],
    out_shape=[
        jax.ShapeDtypeStruct((_N, _D), jnp.float32),
        jax.ShapeDtypeStruct((_N, 16), jnp.float32),
    ],
)


def _kmid_body(s0_ref, s1_ref, g_ref, dis_ref, b_ref, w_ref, out_ref):
    dis = dis_ref[...][:, :1]
    y = dis * (s0_ref[...] + s1_ref[...] + g_ref[...]) + b_ref[...]
    y = jnp.maximum(y, 0.0)
    out_ref[...] = jnp.dot(y, w_ref[...], preferred_element_type=jnp.float32) * dis


_kmid = pl.pallas_call(
    _kmid_body,
    grid=(_GRID,),
    in_specs=[
        pl.BlockSpec((_BR, _D), lambda i: (i, 0)),
        pl.BlockSpec((_BR, _D), lambda i: (i, 0)),
        pl.BlockSpec((_BR, _D), lambda i: (i, 0)),
        pl.BlockSpec((_BR, 16), lambda i: (i, 0)),
        pl.BlockSpec((1, _D), lambda i: (0, 0)),
        pl.BlockSpec((_D, _D), lambda i: (0, 0)),
    ],
    out_specs=pl.BlockSpec((_BR, _D), lambda i: (i, 0)),
    out_shape=jax.ShapeDtypeStruct((_N, _D), jnp.float32),
)


def _khead_body(s0_ref, s1_ref, g_ref, dis_ref, b3_ref,
                w1_ref, c1_ref, w2_ref, c2_ref, w3_ref, c3_ref, out_ref):
    dis = dis_ref[...][:, :1]
    y = dis * (s0_ref[...] + s1_ref[...] + g_ref[...]) + b3_ref[...]
    y = jnp.maximum(y, 0.0)
    y = jnp.dot(y, w1_ref[...], preferred_element_type=jnp.float32) + c1_ref[...]
    y = jnp.maximum(y, 0.0)
    y = jnp.dot(y, w2_ref[...], preferred_element_type=jnp.float32) + c2_ref[...]
    y = jnp.maximum(y, 0.0)
    out_ref[...] = jnp.dot(y, w3_ref[...], preferred_element_type=jnp.float32) + c3_ref[...]


_khead = pl.pallas_call(
    _khead_body,
    grid=(_GRID,),
    in_specs=[
        pl.BlockSpec((_BR, _D), lambda i: (i, 0)),
        pl.BlockSpec((_BR, _D), lambda i: (i, 0)),
        pl.BlockSpec((_BR, _D), lambda i: (i, 0)),
        pl.BlockSpec((_BR, 16), lambda i: (i, 0)),
        pl.BlockSpec((1, _D), lambda i: (0, 0)),
        pl.BlockSpec((_D, _D), lambda i: (0, 0)),
        pl.BlockSpec((1, _D), lambda i: (0, 0)),
        pl.BlockSpec((_D, _D), lambda i: (0, 0)),
        pl.BlockSpec((1, _D), lambda i: (0, 0)),
        pl.BlockSpec((_D, _D), lambda i: (0, 0)),
        pl.BlockSpec((1, _D), lambda i: (0, 0)),
    ],
    out_specs=pl.BlockSpec((_BR, _D), lambda i: (i, 0)),
    out_shape=jax.ShapeDtypeStruct((_N, _D), jnp.float32),
)


def kernel(x, edge_index, TRAIN, W1, b1, W2, b2, W3, b3,
           fcW1, fcb1, fcW2, fcb2, fcW3, fcb3):
    del TRAIN
    src = edge_index[0].reshape(_NW, _NCHUNK, _K)
    dst = edge_index[1].reshape(_NW, _NCHUNK, _K)
    ones16 = jnp.ones((_K, 16), jnp.float32)
    zeros16 = jnp.zeros((_RPT, 16), jnp.float32)
    zerosD = jnp.zeros((_RPT, _D), jnp.float32)

    deg = _deg_pass(dst, ones16, zeros16).reshape(_NC, _NP, 16)[:, :_N]
    g1, dis16 = _k0(x, W1, deg[0], deg[1])

    def _prop(g):
        sp = _prop_pass(g, src, dst, zerosD).reshape(_NC, _NP, _D)
        return sp[0, :_N], sp[1, :_N]

    s0, s1 = _prop(g1)
    g2 = _kmid(s0, s1, g1, dis16, b1.reshape(1, _D), W2)
    s0, s1 = _prop(g2)
    g3 = _kmid(s0, s1, g2, dis16, b2.reshape(1, _D), W3)
    s0, s1 = _prop(g3)

    fcW3p = jnp.pad(fcW3, ((0, 0), (0, _D - _C)))
    fcb3p = jnp.pad(fcb3, (0, _D - _C)).reshape(1, _D)
    y = _khead(s0, s1, g3, dis16, b3.reshape(1, _D),
               fcW1, fcb1.reshape(1, _D), fcW2, fcb2.reshape(1, _D),
               fcW3p, fcb3p)
    return y[:, :_C]
